# Initial kernel scaffold; baseline (speedup 1.0000x reference)
import functools

import jax
import jax.numpy as jnp
from jax import lax
from jax.experimental import pallas as pl
from jax.experimental.pallas import tpu as pltpu
from jax.experimental.pallas import tpu_sc as plsc

N = 10000
E = 320000
F = 128
N_RBF = 100
GAMMA = 10.0
STEP = 0.1
LN2 = 0.6931471805599453

NC = 2
NS = 16
NW = NC * NS
EW = E // NW
K = 100
C = EW // K
RPT = N // NS

BE = 2560
NB = E // BE

_MESH = plsc.VectorSubcoreMesh(core_axis_name="c", subcore_axis_name="s")


def _ssp(t):
    return jnp.logaddexp(t, 0.0) - LN2


def _mm1_body(x_ref, w_ref, b_ref, o_ref):
    o_ref[...] = (
        jnp.dot(x_ref[...], w_ref[...], preferred_element_type=jnp.float32)
        + b_ref[...]
    )


@functools.partial(
    pl.kernel,
    out_type=jax.ShapeDtypeStruct((E,), jnp.float32),
    mesh=_MESH,
    scratch_types=[
        pltpu.VMEM((N,), jnp.float32),
        pltpu.VMEM((N,), jnp.float32),
        pltpu.VMEM((N,), jnp.float32),
        pltpu.VMEM((EW,), jnp.int32),
        pltpu.VMEM((EW,), jnp.int32),
        pltpu.VMEM((EW,), jnp.float32),
    ],
)
def _sc_d2(px_hbm, py_hbm, pz_hbm, src_hbm, dst_hbm, d2_hbm,
           px_v, py_v, pz_v, src_v, dst_v, d2_v):
    cid = lax.axis_index("c")
    sid = lax.axis_index("s")
    wid = sid * NC + cid
    base = wid * EW
    pltpu.sync_copy(px_hbm, px_v)
    pltpu.sync_copy(py_hbm, py_v)
    pltpu.sync_copy(pz_hbm, pz_v)
    pltpu.sync_copy(src_hbm.at[pl.ds(base, EW)], src_v)
    pltpu.sync_copy(dst_hbm.at[pl.ds(base, EW)], dst_v)

    def body(i, carry):
        off = i * 16
        s16 = src_v[pl.ds(off, 16)]
        t16 = dst_v[pl.ds(off, 16)]
        ax = plsc.load_gather(px_v, [s16]) - plsc.load_gather(px_v, [t16])
        ay = plsc.load_gather(py_v, [s16]) - plsc.load_gather(py_v, [t16])
        az = plsc.load_gather(pz_v, [s16]) - plsc.load_gather(pz_v, [t16])
        d2_v[pl.ds(off, 16)] = ax * ax + ay * ay + az * az + 1e-12
        return carry

    lax.fori_loop(0, EW // 16, body, 0)
    pltpu.sync_copy(d2_v, d2_hbm.at[pl.ds(base, EW)])


def _filter_body(d2_ref, wf1t_ref, bf1_ref, wf2_ref, bf2_ref, o_ref):
    d = jnp.sqrt(d2_ref[...])
    mu = lax.broadcasted_iota(jnp.float32, (F, 1), 0) * STEP
    diff = d[None, :] - mu
    rbft = jnp.exp(diff * diff * (-GAMMA))
    h = _ssp(
        jnp.dot(wf1t_ref[...], rbft, preferred_element_type=jnp.float32)
        + bf1_ref[...]
    )
    w = lax.dot_general(
        h, wf2_ref[...], (((0,), (0,)), ((), ())),
        preferred_element_type=jnp.float32,
    )
    o_ref[...] = w + bf2_ref[...]


@functools.partial(
    pl.kernel,
    out_type=jax.ShapeDtypeStruct((NC, N, F), jnp.float32),
    mesh=_MESH,
    scratch_types=[
        pltpu.VMEM((C, K), jnp.int32),
        pltpu.VMEM((C, K), jnp.int32),
        pltpu.VMEM((K, F), jnp.float32),
        pltpu.VMEM((K, F), jnp.float32),
        pltpu.VMEM_SHARED((N, F), jnp.float32),
        pltpu.SemaphoreType.DMA,
    ],
)
def _sc_gms(m_hbm, w_hbm, src_hbm, dst_hbm, zero_hbm, vp_hbm,
            src_v, dst_v, rows_v, w_v, vacc, sem):
    cid = lax.axis_index("c")
    sid = lax.axis_index("s")
    wid = sid * NC + cid
    pltpu.sync_copy(src_hbm.at[wid], src_v)
    pltpu.sync_copy(dst_hbm.at[wid], dst_v)
    pltpu.sync_copy(zero_hbm, vacc.at[pl.ds(sid * RPT, RPT)])
    plsc.subcore_barrier()
    gbase = wid * EW

    def chunk(j, carry):
        pltpu.async_copy(m_hbm.at[src_v.at[j]], rows_v, sem).wait()
        pltpu.sync_copy(w_hbm.at[pl.ds(gbase + j * K, K)], w_v)

        def mul(r, c2):
            for cc in range(F // 16):
                sl = pl.ds(cc * 16, 16)
                rows_v[r, sl] = rows_v[r, sl] * w_v[r, sl]
            return c2

        lax.fori_loop(0, K, mul, 0)
        pltpu.sync_copy(rows_v, vacc.at[dst_v.at[j]], add=True)
        return carry

    lax.fori_loop(0, C, chunk, 0)
    plsc.subcore_barrier()
    pltpu.sync_copy(vacc.at[pl.ds(sid * RPT, RPT)],
                    vp_hbm.at[cid, pl.ds(sid * RPT, RPT)])


def _out_body(x_ref, vp_ref, w2_ref, b2_ref, w3_ref, b3_ref, o_ref):
    v = vp_ref[0] + vp_ref[1]
    t = _ssp(
        jnp.dot(v, w2_ref[...], preferred_element_type=jnp.float32)
        + b2_ref[...]
    )
    o_ref[...] = (
        x_ref[...]
        + jnp.dot(t, w3_ref[...], preferred_element_type=jnp.float32)
        + b3_ref[...]
    )


def kernel(x, edge_index, z, position, W1, b1, Wf1, bf1, Wf2, bf2, W2, b2, W3, b3):
    del z
    src = edge_index[0]
    dst = edge_index[1]
    px = position[:, 0]
    py = position[:, 1]
    pz = position[:, 2]

    m = pl.pallas_call(
        _mm1_body,
        out_shape=jax.ShapeDtypeStruct((N, F), jnp.float32),
    )(x, W1, b1.reshape(1, F))

    d2 = _sc_d2(px, py, pz, src, dst)

    wf1t = jnp.zeros((F, F), jnp.float32).at[:, :N_RBF].set(Wf1.T)
    w = pl.pallas_call(
        _filter_body,
        grid=(NB,),
        in_specs=[
            pl.BlockSpec((BE,), lambda i: (i,)),
            pl.BlockSpec((F, F), lambda i: (0, 0)),
            pl.BlockSpec((F, 1), lambda i: (0, 0)),
            pl.BlockSpec((F, F), lambda i: (0, 0)),
            pl.BlockSpec((1, F), lambda i: (0, 0)),
        ],
        out_specs=pl.BlockSpec((BE, F), lambda i: (i, 0)),
        out_shape=jax.ShapeDtypeStruct((E, F), jnp.float32),
    )(d2, wf1t, bf1.reshape(F, 1), Wf2, bf2.reshape(1, F))

    src3 = src.reshape(NW, C, K)
    dst3 = dst.reshape(NW, C, K)
    zero = jnp.zeros((RPT, F), jnp.float32)
    vp = _sc_gms(m, w, src3, dst3, zero)

    out = pl.pallas_call(
        _out_body,
        out_shape=jax.ShapeDtypeStruct((N, F), jnp.float32),
    )(x, vp, W2, b2.reshape(1, F), W3, b3.reshape(1, F))
    return out


# speedup vs baseline: 2.9962x; 2.9962x over previous
import functools

import jax
import jax.numpy as jnp
from jax import lax
from jax.experimental import pallas as pl
from jax.experimental.pallas import tpu as pltpu
from jax.experimental.pallas import tpu_sc as plsc

N = 10000
E = 320000
F = 128
N_RBF = 100
GAMMA = 10.0
STEP = 0.1
LN2 = 0.6931471805599453

NC = 2
NS = 16
NW = NC * NS
EW = E // NW
K = 80
C = EW // K

NPH = 5120
TRASH = 8
PPT = NPH // NS

BE = 512
NB = E // BE


@functools.lru_cache(maxsize=None)
def _mesh():
    return plsc.VectorSubcoreMesh(
        core_axis_name="c", subcore_axis_name="s",
        num_cores=NC, num_subcores=NS,
    )


def _ssp(t):
    return jnp.logaddexp(t, 0.0) - LN2


def _mm1_body(x_ref, w_ref, b_ref, o_ref):
    o_ref[...] = (
        jnp.dot(x_ref[...], w_ref[...], preferred_element_type=jnp.float32)
        + b_ref[...]
    )


@functools.lru_cache(maxsize=None)
def _sc_d2_kernel():
    return functools.partial(
        pl.kernel,
        out_type=jax.ShapeDtypeStruct((E,), jnp.float32),
        mesh=_mesh(),
        compiler_params=pltpu.CompilerParams(needs_layout_passes=False),
        scratch_types=[
            pltpu.VMEM((N,), jnp.float32),
            pltpu.VMEM((N,), jnp.float32),
            pltpu.VMEM((N,), jnp.float32),
            pltpu.VMEM((EW,), jnp.int32),
            pltpu.VMEM((EW,), jnp.int32),
            pltpu.VMEM((EW,), jnp.float32),
        ],
    )(_sc_d2)


def _sc_d2(px_hbm, py_hbm, pz_hbm, src_hbm, dst_hbm, d2_hbm,
           px_v, py_v, pz_v, src_v, dst_v, d2_v):
    cid = lax.axis_index("c")
    sid = lax.axis_index("s")
    wid = sid * NC + cid
    base = wid * EW
    pltpu.sync_copy(px_hbm, px_v)
    pltpu.sync_copy(py_hbm, py_v)
    pltpu.sync_copy(pz_hbm, pz_v)
    pltpu.sync_copy(src_hbm.at[pl.ds(base, EW)], src_v)
    pltpu.sync_copy(dst_hbm.at[pl.ds(base, EW)], dst_v)

    def body(i, carry):
        off = i * 16
        s16 = src_v[pl.ds(off, 16)]
        t16 = dst_v[pl.ds(off, 16)]
        ax = plsc.load_gather(px_v, [s16]) - plsc.load_gather(px_v, [t16])
        ay = plsc.load_gather(py_v, [s16]) - plsc.load_gather(py_v, [t16])
        az = plsc.load_gather(pz_v, [s16]) - plsc.load_gather(pz_v, [t16])
        d2_v[pl.ds(off, 16)] = ax * ax + ay * ay + az * az + 1e-12
        return carry

    lax.fori_loop(0, EW // 16, body, 0)
    pltpu.sync_copy(d2_v, d2_hbm.at[pl.ds(base, EW)])


def _filter_body(d2_ref, wf1t_ref, bf1_ref, wf2_ref, bf2_ref, o_ref):
    d = jnp.sqrt(d2_ref[...])
    mu = lax.broadcasted_iota(jnp.int32, (F, 1), 0).astype(jnp.float32) * STEP
    diff = d[None, :] - mu
    rbft = jnp.exp(diff * diff * (-GAMMA))
    h = _ssp(
        jnp.dot(wf1t_ref[...], rbft, preferred_element_type=jnp.float32)
        + bf1_ref[...]
    )
    w = lax.dot_general(
        h, wf2_ref[...], (((0,), (0,)), ((), ())),
        preferred_element_type=jnp.float32,
    )
    o_ref[...] = w + bf2_ref[...]


@functools.lru_cache(maxsize=None)
def _sc_gms_kernel():
    return functools.partial(
        pl.kernel,
        out_type=jax.ShapeDtypeStruct((NC, N, F), jnp.float32),
        mesh=_mesh(),
        compiler_params=pltpu.CompilerParams(needs_layout_passes=False),
        scratch_types=[
            pltpu.VMEM((EW,), jnp.int32),
            pltpu.VMEM((EW,), jnp.int32),
            pltpu.VMEM((2, K), jnp.int32),
            pltpu.VMEM((K, F), jnp.float32),
            pltpu.VMEM((K, F), jnp.float32),
            pltpu.VMEM_SHARED((NPH + TRASH, F), jnp.float32),
            pltpu.SemaphoreType.DMA,
        ],
    )(_sc_gms)


def _sc_gms(m_hbm, w_hbm, edges_hbm, vp_hbm,
            src_v, dst_v, ixs_v, rows_v, w_v, vacc, sem):
    cid = lax.axis_index("c")
    sid = lax.axis_index("s")
    wid = sid * NC + cid
    pltpu.sync_copy(edges_hbm.at[0, wid], src_v)
    pltpu.sync_copy(edges_hbm.at[1, wid], dst_v)
    gbase = wid * EW

    for p in range(N // NPH + (1 if N % NPH else 0)):
        base = p * NPH
        nph_real = min(NPH, N - base)

        def zrow(r, carry):
            for cc in range(F // 16):
                rows_v[r, pl.ds(cc * 16, 16)] = jnp.zeros((16,), jnp.float32)
            return carry

        lax.fori_loop(0, K, zrow, 0)

        def zcopy(t, carry):
            pltpu.sync_copy(rows_v.at[pl.ds(0, 40)],
                            vacc.at[pl.ds(sid * PPT + t * 40, 40)])
            return carry

        lax.fori_loop(0, PPT // 40, zcopy, 0)

        @pl.when(sid == NS - 1)
        def _zero_trash():
            pltpu.sync_copy(rows_v.at[pl.ds(0, TRASH)],
                            vacc.at[pl.ds(NPH, TRASH)])

        plsc.subcore_barrier()

        def chunk(j, carry):
            gather = pltpu.async_copy(
                m_hbm.at[src_v.at[pl.ds(j * K, K)]], rows_v, sem)
            pltpu.sync_copy(w_hbm.at[pl.ds(gbase + j * K, K)], w_v)

            for g in range(K // 16):
                sl = pl.ds(g * 16, 16)
                rel = dst_v[pl.ds(j * K + g * 16, 16)] - base
                inr = jnp.logical_and(rel >= 0, rel < NPH)
                low = jnp.bitwise_and(rel, TRASH - 1)
                ixs_v[0, sl] = jnp.where(inr, rel, NPH + low)

            gather.wait()

            def mul(r, c2):
                for cc in range(F // 16):
                    sl = pl.ds(cc * 16, 16)
                    rows_v[r, sl] = rows_v[r, sl] * w_v[r, sl]
                return c2

            lax.fori_loop(0, K, mul, 0)
            pltpu.sync_copy(rows_v, vacc.at[ixs_v.at[0]], add=True)
            return carry

        lax.fori_loop(0, C, chunk, 0)
        plsc.subcore_barrier()

        full_tiles = nph_real // PPT
        rem = nph_real - full_tiles * PPT

        @pl.when(sid < full_tiles)
        def _dump_full():
            pltpu.sync_copy(vacc.at[pl.ds(sid * PPT, PPT)],
                            vp_hbm.at[cid, pl.ds(base + sid * PPT, PPT)])

        if rem:
            @pl.when(sid == full_tiles)
            def _dump_rem():
                pltpu.sync_copy(
                    vacc.at[pl.ds(full_tiles * PPT, rem)],
                    vp_hbm.at[cid, pl.ds(base + full_tiles * PPT, rem)])

        plsc.subcore_barrier()


def _out_body(x_ref, vp_ref, w2_ref, b2_ref, w3_ref, b3_ref, o_ref):
    v = vp_ref[0] + vp_ref[1]
    t = _ssp(
        jnp.dot(v, w2_ref[...], preferred_element_type=jnp.float32)
        + b2_ref[...]
    )
    o_ref[...] = (
        x_ref[...]
        + jnp.dot(t, w3_ref[...], preferred_element_type=jnp.float32)
        + b3_ref[...]
    )


def kernel(x, edge_index, z, position, W1, b1, Wf1, bf1, Wf2, bf2, W2, b2, W3, b3):
    del z
    src = edge_index[0]
    dst = edge_index[1]
    px = position[:, 0]
    py = position[:, 1]
    pz = position[:, 2]

    m = pl.pallas_call(
        _mm1_body,
        out_shape=jax.ShapeDtypeStruct((N, F), jnp.float32),
    )(x, W1, b1.reshape(1, F))

    d2 = _sc_d2_kernel()(px, py, pz, src, dst)

    wf1t = jnp.zeros((F, F), jnp.float32).at[:, :N_RBF].set(Wf1.T)
    w = pl.pallas_call(
        _filter_body,
        grid=(NB,),
        in_specs=[
            pl.BlockSpec((BE,), lambda i: (i,)),
            pl.BlockSpec((F, F), lambda i: (0, 0)),
            pl.BlockSpec((F, 1), lambda i: (0, 0)),
            pl.BlockSpec((F, F), lambda i: (0, 0)),
            pl.BlockSpec((1, F), lambda i: (0, 0)),
        ],
        out_specs=pl.BlockSpec((BE, F), lambda i: (i, 0)),
        out_shape=jax.ShapeDtypeStruct((E, F), jnp.float32),
    )(d2, wf1t, bf1.reshape(F, 1), Wf2, bf2.reshape(1, F))

    edges = edge_index.reshape(2, NW, EW)
    vp = _sc_gms_kernel()(m, w, edges)

    out = pl.pallas_call(
        _out_body,
        out_shape=jax.ShapeDtypeStruct((N, F), jnp.float32),
    )(x, vp, W2, b2.reshape(1, F), W3, b3.reshape(1, F))
    return out


# speedup vs baseline: 3.6816x; 1.2287x over previous
import functools

import jax
import jax.numpy as jnp
from jax import lax
from jax.experimental import pallas as pl
from jax.experimental.pallas import tpu as pltpu
from jax.experimental.pallas import tpu_sc as plsc

N = 10000
E = 320000
F = 128
N_RBF = 100
GAMMA = 10.0
STEP = 0.1
LN2 = 0.6931471805599453

NC = 2
NS = 16
NW = NC * NS
EW = E // NW
K = 80
C = EW // K

NPH = 5120
TRASH = 8
PPT = NPH // NS

BE = 2560
NB = E // BE


@functools.lru_cache(maxsize=None)
def _mesh():
    return plsc.VectorSubcoreMesh(
        core_axis_name="c", subcore_axis_name="s",
        num_cores=NC, num_subcores=NS,
    )


def _ssp(t):
    return jnp.logaddexp(t, 0.0) - LN2


def _mm1_body(x_ref, w_ref, b_ref, o_ref):
    o_ref[...] = (
        jnp.dot(x_ref[...], w_ref[...], preferred_element_type=jnp.float32)
        + b_ref[...]
    )


@functools.lru_cache(maxsize=None)
def _sc_d2_kernel():
    return functools.partial(
        pl.kernel,
        out_type=jax.ShapeDtypeStruct((E,), jnp.float32),
        mesh=_mesh(),
        compiler_params=pltpu.CompilerParams(needs_layout_passes=False),
        scratch_types=[
            pltpu.VMEM((N,), jnp.float32),
            pltpu.VMEM((N,), jnp.float32),
            pltpu.VMEM((N,), jnp.float32),
            pltpu.VMEM((EW,), jnp.int32),
            pltpu.VMEM((EW,), jnp.int32),
            pltpu.VMEM((EW,), jnp.float32),
        ],
    )(_sc_d2)


def _sc_d2(px_hbm, py_hbm, pz_hbm, src_hbm, dst_hbm, d2_hbm,
           px_v, py_v, pz_v, src_v, dst_v, d2_v):
    cid = lax.axis_index("c")
    sid = lax.axis_index("s")
    wid = sid * NC + cid
    base = wid * EW
    pltpu.sync_copy(px_hbm, px_v)
    pltpu.sync_copy(py_hbm, py_v)
    pltpu.sync_copy(pz_hbm, pz_v)
    pltpu.sync_copy(src_hbm.at[pl.ds(base, EW)], src_v)
    pltpu.sync_copy(dst_hbm.at[pl.ds(base, EW)], dst_v)

    def body(i, carry):
        off = i * 16
        s16 = src_v[pl.ds(off, 16)]
        t16 = dst_v[pl.ds(off, 16)]
        ax = plsc.load_gather(px_v, [s16]) - plsc.load_gather(px_v, [t16])
        ay = plsc.load_gather(py_v, [s16]) - plsc.load_gather(py_v, [t16])
        az = plsc.load_gather(pz_v, [s16]) - plsc.load_gather(pz_v, [t16])
        d2_v[pl.ds(off, 16)] = ax * ax + ay * ay + az * az + 1e-12
        return carry

    lax.fori_loop(0, EW // 16, body, 0)
    pltpu.sync_copy(d2_v, d2_hbm.at[pl.ds(base, EW)])


def _filter_body(d2_ref, wf1t_ref, bf1_ref, wf2_ref, bf2_ref, o_ref):
    d = jnp.sqrt(d2_ref[0])
    mu = lax.broadcasted_iota(jnp.int32, (F, 1), 0).astype(jnp.float32) * STEP
    diff = d - mu
    rbft = jnp.exp(diff * diff * (-GAMMA))
    h = _ssp(
        jnp.dot(wf1t_ref[...], rbft, preferred_element_type=jnp.float32)
        + bf1_ref[...]
    )
    w = lax.dot_general(
        h, wf2_ref[...], (((0,), (0,)), ((), ())),
        preferred_element_type=jnp.float32,
    )
    o_ref[...] = w + bf2_ref[...]


@functools.lru_cache(maxsize=None)
def _sc_gms_kernel():
    return functools.partial(
        pl.kernel,
        out_type=jax.ShapeDtypeStruct((NC, N, F), jnp.float32),
        mesh=_mesh(),
        compiler_params=pltpu.CompilerParams(needs_layout_passes=False),
        scratch_types=[
            pltpu.VMEM((EW,), jnp.int32),
            pltpu.VMEM((EW,), jnp.int32),
            pltpu.VMEM((2, K), jnp.int32),
            pltpu.VMEM((K, F), jnp.float32),
            pltpu.VMEM((K, F), jnp.float32),
            pltpu.VMEM((K, F), jnp.float32),
            pltpu.VMEM((K, F), jnp.float32),
            pltpu.VMEM_SHARED((NPH + TRASH, F), jnp.float32),
            pltpu.SemaphoreType.DMA,
            pltpu.SemaphoreType.DMA,
            pltpu.SemaphoreType.DMA,
            pltpu.SemaphoreType.DMA,
        ],
    )(_sc_gms)


def _sc_gms(m_hbm, w_hbm, edges_hbm, vp_hbm,
            src_v, dst_v, ixs_v, rows_0, rows_1, w_0, w_1, vacc,
            gsem_0, gsem_1, wsem_0, wsem_1):
    cid = lax.axis_index("c")
    sid = lax.axis_index("s")
    wid = sid * NC + cid
    pltpu.sync_copy(edges_hbm.at[0, wid], src_v)
    pltpu.sync_copy(edges_hbm.at[1, wid], dst_v)
    gbase = wid * EW

    for p in range(N // NPH + (1 if N % NPH else 0)):
        base = p * NPH
        nph_real = min(NPH, N - base)

        def zrow(r, carry):
            for cc in range(F // 16):
                rows_0[r, pl.ds(cc * 16, 16)] = jnp.zeros((16,), jnp.float32)
            return carry

        lax.fori_loop(0, K, zrow, 0)

        def zcopy(t, carry):
            pltpu.sync_copy(rows_0.at[pl.ds(0, 40)],
                            vacc.at[pl.ds(sid * PPT + t * 40, 40)])
            return carry

        lax.fori_loop(0, PPT // 40, zcopy, 0)

        @pl.when(sid == NS - 1)
        def _zero_trash():
            pltpu.sync_copy(rows_0.at[pl.ds(0, TRASH)],
                            vacc.at[pl.ds(NPH, TRASH)])

        plsc.subcore_barrier()

        def chunk(j, carry):
            gather = pltpu.async_copy(
                m_hbm.at[src_v.at[pl.ds(j * K, K)]], rows_0, gsem_0)
            pltpu.sync_copy(w_hbm.at[pl.ds(gbase + j * K, K)], w_0)

            for g in range(K // 16):
                sl = pl.ds(g * 16, 16)
                rel = dst_v[pl.ds(j * K + g * 16, 16)] - base
                inr = jnp.logical_and(rel >= 0, rel < NPH)
                low = jnp.bitwise_and(rel, TRASH - 1)
                ixs_v[0, sl] = jnp.where(inr, rel, NPH + low)

            gather.wait()

            def mul(r, c2):
                for cc in range(F // 16):
                    sl = pl.ds(cc * 16, 16)
                    rows_0[r, sl] = rows_0[r, sl] * w_0[r, sl]
                return c2

            lax.fori_loop(0, K, mul, 0)
            pltpu.sync_copy(rows_0, vacc.at[ixs_v.at[0]], add=True)
            return carry

        lax.fori_loop(0, C, chunk, 0)
        plsc.subcore_barrier()

        full_tiles = nph_real // PPT
        rem = nph_real - full_tiles * PPT

        @pl.when(sid < full_tiles)
        def _dump_full():
            pltpu.sync_copy(vacc.at[pl.ds(sid * PPT, PPT)],
                            vp_hbm.at[cid, pl.ds(base + sid * PPT, PPT)])

        if rem:
            @pl.when(sid == full_tiles)
            def _dump_rem():
                pltpu.sync_copy(
                    vacc.at[pl.ds(full_tiles * PPT, rem)],
                    vp_hbm.at[cid, pl.ds(base + full_tiles * PPT, rem)])

        plsc.subcore_barrier()


def _out_body(x_ref, vp_ref, w2_ref, b2_ref, w3_ref, b3_ref, o_ref):
    v = vp_ref[0] + vp_ref[1]
    t = _ssp(
        jnp.dot(v, w2_ref[...], preferred_element_type=jnp.float32)
        + b2_ref[...]
    )
    o_ref[...] = (
        x_ref[...]
        + jnp.dot(t, w3_ref[...], preferred_element_type=jnp.float32)
        + b3_ref[...]
    )


def kernel(x, edge_index, z, position, W1, b1, Wf1, bf1, Wf2, bf2, W2, b2, W3, b3):
    del z
    src = edge_index[0]
    dst = edge_index[1]
    px = position[:, 0]
    py = position[:, 1]
    pz = position[:, 2]

    m = pl.pallas_call(
        _mm1_body,
        out_shape=jax.ShapeDtypeStruct((N, F), jnp.float32),
    )(x, W1, b1.reshape(1, F))

    d2 = _sc_d2_kernel()(px, py, pz, src, dst)

    wf1t = jnp.zeros((F, F), jnp.float32).at[:, :N_RBF].set(Wf1.T)
    w = pl.pallas_call(
        _filter_body,
        grid=(NB,),
        in_specs=[
            pl.BlockSpec((1, 1, BE), lambda i: (i, 0, 0)),
            pl.BlockSpec((F, F), lambda i: (0, 0)),
            pl.BlockSpec((F, 1), lambda i: (0, 0)),
            pl.BlockSpec((F, F), lambda i: (0, 0)),
            pl.BlockSpec((1, F), lambda i: (0, 0)),
        ],
        out_specs=pl.BlockSpec((BE, F), lambda i: (i, 0)),
        out_shape=jax.ShapeDtypeStruct((E, F), jnp.float32),
    )(d2.reshape(NB, 1, BE), wf1t, bf1.reshape(F, 1), Wf2, bf2.reshape(1, F))

    edges = edge_index.reshape(2, NW, EW)
    vp = _sc_gms_kernel()(m, w, edges)

    out = pl.pallas_call(
        _out_body,
        out_shape=jax.ShapeDtypeStruct((N, F), jnp.float32),
    )(x, vp, W2, b2.reshape(1, F), W3, b3.reshape(1, F))
    return out


# speedup vs baseline: 5.3417x; 1.4509x over previous
import functools

import jax
import jax.numpy as jnp
from jax import lax
from jax.experimental import pallas as pl
from jax.experimental.pallas import tpu as pltpu
from jax.experimental.pallas import tpu_sc as plsc

N = 10000
E = 320000
F = 128
N_RBF = 100
GAMMA = 10.0
STEP = 0.1
LN2 = 0.6931471805599453

NC = 2
NS = 16
NW = NC * NS
EW = E // NW
K = 80
C = EW // K

NPH = 5120
TRASH = 8
PPT = NPH // NS

BE = 2560
NB = E // BE


@functools.lru_cache(maxsize=None)
def _mesh():
    return plsc.VectorSubcoreMesh(
        core_axis_name="c", subcore_axis_name="s",
        num_cores=NC, num_subcores=NS,
    )


def _ssp(t):
    return jnp.logaddexp(t, 0.0) - LN2


def _mm1_body(x_ref, w_ref, b_ref, o_ref):
    o_ref[...] = (
        jnp.dot(x_ref[...], w_ref[...], preferred_element_type=jnp.float32)
        + b_ref[...]
    )


@functools.lru_cache(maxsize=None)
def _sc_d2_kernel():
    return functools.partial(
        pl.kernel,
        out_type=jax.ShapeDtypeStruct((E,), jnp.float32),
        mesh=_mesh(),
        compiler_params=pltpu.CompilerParams(needs_layout_passes=False),
        scratch_types=[
            pltpu.VMEM((N,), jnp.float32),
            pltpu.VMEM((N,), jnp.float32),
            pltpu.VMEM((N,), jnp.float32),
            pltpu.VMEM((EW,), jnp.int32),
            pltpu.VMEM((EW,), jnp.int32),
            pltpu.VMEM((EW,), jnp.float32),
        ],
    )(_sc_d2)


def _sc_d2(px_hbm, py_hbm, pz_hbm, src_hbm, dst_hbm, d2_hbm,
           px_v, py_v, pz_v, src_v, dst_v, d2_v):
    cid = lax.axis_index("c")
    sid = lax.axis_index("s")
    wid = sid * NC + cid
    base = wid * EW
    pltpu.sync_copy(px_hbm, px_v)
    pltpu.sync_copy(py_hbm, py_v)
    pltpu.sync_copy(pz_hbm, pz_v)
    pltpu.sync_copy(src_hbm.at[pl.ds(base, EW)], src_v)
    pltpu.sync_copy(dst_hbm.at[pl.ds(base, EW)], dst_v)

    def body(i, carry):
        off = i * 16
        s16 = src_v[pl.ds(off, 16)]
        t16 = dst_v[pl.ds(off, 16)]
        ax = plsc.load_gather(px_v, [s16]) - plsc.load_gather(px_v, [t16])
        ay = plsc.load_gather(py_v, [s16]) - plsc.load_gather(py_v, [t16])
        az = plsc.load_gather(pz_v, [s16]) - plsc.load_gather(pz_v, [t16])
        d2_v[pl.ds(off, 16)] = ax * ax + ay * ay + az * az + 1e-12
        return carry

    lax.fori_loop(0, EW // 16, body, 0)
    pltpu.sync_copy(d2_v, d2_hbm.at[pl.ds(base, EW)])


def _filter_body(d2_ref, wf1t_ref, bf1_ref, wf2_ref, bf2_ref, o_ref):
    d = jnp.sqrt(d2_ref[0])
    mu = lax.broadcasted_iota(jnp.int32, (F, 1), 0).astype(jnp.float32) * STEP
    diff = d - mu
    rbft = jnp.exp(diff * diff * (-GAMMA))
    h = _ssp(
        jnp.dot(wf1t_ref[...], rbft, preferred_element_type=jnp.float32)
        + bf1_ref[...]
    )
    w = lax.dot_general(
        h, wf2_ref[...], (((0,), (0,)), ((), ())),
        preferred_element_type=jnp.float32,
    )
    o_ref[...] = w + bf2_ref[...]


@functools.lru_cache(maxsize=None)
def _sc_gms_kernel():
    return functools.partial(
        pl.kernel,
        out_type=jax.ShapeDtypeStruct((NC, N, F), jnp.float32),
        mesh=_mesh(),
        compiler_params=pltpu.CompilerParams(needs_layout_passes=False),
        scratch_types=[
            pltpu.VMEM((EW,), jnp.int32),
            pltpu.VMEM((EW,), jnp.int32),
            pltpu.VMEM((2, K), jnp.int32),
            pltpu.VMEM((K, F), jnp.float32),
            pltpu.VMEM((K, F), jnp.float32),
            pltpu.VMEM((K, F), jnp.float32),
            pltpu.VMEM((K, F), jnp.float32),
            pltpu.VMEM_SHARED((NPH + TRASH, F), jnp.float32),
            pltpu.SemaphoreType.DMA,
            pltpu.SemaphoreType.DMA,
            pltpu.SemaphoreType.DMA,
            pltpu.SemaphoreType.DMA,
        ],
    )(_sc_gms)


def _sc_gms(m_hbm, w_hbm, edges_hbm, vp_hbm,
            src_v, dst_v, ixs_v, rows_0, rows_1, w_0, w_1, vacc,
            gsem_0, gsem_1, wsem_0, wsem_1):
    cid = lax.axis_index("c")
    sid = lax.axis_index("s")
    wid = sid * NC + cid
    pltpu.sync_copy(edges_hbm.at[0, wid], src_v)
    pltpu.sync_copy(edges_hbm.at[1, wid], dst_v)
    gbase = wid * EW

    for p in range(N // NPH + (1 if N % NPH else 0)):
        base = p * NPH
        nph_real = min(NPH, N - base)

        def zrow(r, carry):
            for cc in range(F // 16):
                rows_0[r, pl.ds(cc * 16, 16)] = jnp.zeros((16,), jnp.float32)
            return carry

        lax.fori_loop(0, K, zrow, 0)

        def zcopy(t, carry):
            pltpu.sync_copy(rows_0.at[pl.ds(0, 40)],
                            vacc.at[pl.ds(sid * PPT + t * 40, 40)])
            return carry

        lax.fori_loop(0, PPT // 40, zcopy, 0)

        @pl.when(sid == NS - 1)
        def _zero_trash():
            pltpu.sync_copy(rows_0.at[pl.ds(0, TRASH)],
                            vacc.at[pl.ds(NPH, TRASH)])

        plsc.subcore_barrier()

        rows_b = (rows_0, rows_1)
        w_b = (w_0, w_1)
        gsem_b = (gsem_0, gsem_1)
        wsem_b = (wsem_0, wsem_1)

        def _gather_desc(j, b):
            return pltpu.make_async_copy(
                m_hbm.at[src_v.at[pl.ds(j * K, K)]], rows_b[b], gsem_b[b])

        def _w_desc(j, b):
            return pltpu.make_async_copy(
                w_hbm.at[pl.ds(gbase + j * K, K)], w_b[b], wsem_b[b])

        def _start(j, b):
            _gather_desc(j, b).start()
            _w_desc(j, b).start()

        def _process(j, b):
            for g in range(K // 16):
                sl = pl.ds(g * 16, 16)
                rel = dst_v[pl.ds(j * K + g * 16, 16)] - base
                inr = jnp.logical_and(rel >= 0, rel < NPH)
                low = jnp.bitwise_and(rel, TRASH - 1)
                ixs_v[b, sl] = jnp.where(inr, rel, NPH + low)

            _gather_desc(j, b).wait()
            _w_desc(j, b).wait()

            def mul(r, c2):
                for cc in range(F // 16):
                    sl = pl.ds(cc * 16, 16)
                    rows_b[b][r, sl] = rows_b[b][r, sl] * w_b[b][r, sl]
                return c2

            lax.fori_loop(0, K, mul, 0)
            pltpu.sync_copy(rows_b[b], vacc.at[ixs_v.at[b]], add=True)

        _start(0, 0)
        _start(1, 1)

        def chunk_pair(jj, carry):
            j0 = jj * 2
            _process(j0, 0)
            _start(j0 + 2, 0)
            _process(j0 + 1, 1)

            @pl.when(jj < C // 2 - 1)
            def _next():
                _start(j0 + 3, 1)

            return carry

        lax.fori_loop(0, C // 2, chunk_pair, 0)
        _process(C - 1, 0)
        plsc.subcore_barrier()

        full_tiles = nph_real // PPT
        rem = nph_real - full_tiles * PPT

        @pl.when(sid < full_tiles)
        def _dump_full():
            pltpu.sync_copy(vacc.at[pl.ds(sid * PPT, PPT)],
                            vp_hbm.at[cid, pl.ds(base + sid * PPT, PPT)])

        if rem:
            @pl.when(sid == full_tiles)
            def _dump_rem():
                pltpu.sync_copy(
                    vacc.at[pl.ds(full_tiles * PPT, rem)],
                    vp_hbm.at[cid, pl.ds(base + full_tiles * PPT, rem)])

        plsc.subcore_barrier()


def _out_body(x_ref, vp_ref, w2_ref, b2_ref, w3_ref, b3_ref, o_ref):
    v = vp_ref[0] + vp_ref[1]
    t = _ssp(
        jnp.dot(v, w2_ref[...], preferred_element_type=jnp.float32)
        + b2_ref[...]
    )
    o_ref[...] = (
        x_ref[...]
        + jnp.dot(t, w3_ref[...], preferred_element_type=jnp.float32)
        + b3_ref[...]
    )


def kernel(x, edge_index, z, position, W1, b1, Wf1, bf1, Wf2, bf2, W2, b2, W3, b3):
    del z
    src = edge_index[0]
    dst = edge_index[1]
    px = position[:, 0]
    py = position[:, 1]
    pz = position[:, 2]

    m = pl.pallas_call(
        _mm1_body,
        out_shape=jax.ShapeDtypeStruct((N, F), jnp.float32),
    )(x, W1, b1.reshape(1, F))

    d2 = _sc_d2_kernel()(px, py, pz, src, dst)

    wf1t = jnp.zeros((F, F), jnp.float32).at[:, :N_RBF].set(Wf1.T)
    w = pl.pallas_call(
        _filter_body,
        grid=(NB,),
        in_specs=[
            pl.BlockSpec((1, 1, BE), lambda i: (i, 0, 0)),
            pl.BlockSpec((F, F), lambda i: (0, 0)),
            pl.BlockSpec((F, 1), lambda i: (0, 0)),
            pl.BlockSpec((F, F), lambda i: (0, 0)),
            pl.BlockSpec((1, F), lambda i: (0, 0)),
        ],
        out_specs=pl.BlockSpec((BE, F), lambda i: (i, 0)),
        out_shape=jax.ShapeDtypeStruct((E, F), jnp.float32),
    )(d2.reshape(NB, 1, BE), wf1t, bf1.reshape(F, 1), Wf2, bf2.reshape(1, F))

    edges = edge_index.reshape(2, NW, EW)
    vp = _sc_gms_kernel()(m, w, edges)

    out = pl.pallas_call(
        _out_body,
        out_shape=jax.ShapeDtypeStruct((N, F), jnp.float32),
    )(x, vp, W2, b2.reshape(1, F), W3, b3.reshape(1, F))
    return out


# speedup vs baseline: 5.9217x; 1.1086x over previous
import functools

import jax
import jax.numpy as jnp
from jax import lax
from jax.experimental import pallas as pl
from jax.experimental.pallas import tpu as pltpu
from jax.experimental.pallas import tpu_sc as plsc

N = 10000
E = 320000
F = 128
N_RBF = 100
GAMMA = 10.0
STEP = 0.1
LN2 = 0.6931471805599453

NC = 2
NS = 16
NW = NC * NS
EW = E // NW
K = 80
C = EW // K

NPH = 5120
TRASH = 8
PPT = NPH // NS

BE = 2560
NB = E // BE


@functools.lru_cache(maxsize=None)
def _mesh():
    return plsc.VectorSubcoreMesh(
        core_axis_name="c", subcore_axis_name="s",
        num_cores=NC, num_subcores=NS,
    )


def _ssp(t):
    return jnp.logaddexp(t, 0.0) - LN2


def _mm1_body(x_ref, w_ref, b_ref, o_ref):
    o_ref[...] = (
        jnp.dot(x_ref[...], w_ref[...], preferred_element_type=jnp.float32)
        + b_ref[...]
    )


@functools.lru_cache(maxsize=None)
def _sc_d2_kernel():
    return functools.partial(
        pl.kernel,
        out_type=jax.ShapeDtypeStruct((E,), jnp.float32),
        mesh=_mesh(),
        compiler_params=pltpu.CompilerParams(needs_layout_passes=False),
        scratch_types=[
            pltpu.VMEM((N,), jnp.float32),
            pltpu.VMEM((N,), jnp.float32),
            pltpu.VMEM((N,), jnp.float32),
            pltpu.VMEM((EW,), jnp.int32),
            pltpu.VMEM((EW,), jnp.int32),
            pltpu.VMEM((EW,), jnp.float32),
        ],
    )(_sc_d2)


def _sc_d2(px_hbm, py_hbm, pz_hbm, src_hbm, dst_hbm, d2_hbm,
           px_v, py_v, pz_v, src_v, dst_v, d2_v):
    cid = lax.axis_index("c")
    sid = lax.axis_index("s")
    wid = sid * NC + cid
    base = wid * EW
    pltpu.sync_copy(px_hbm, px_v)
    pltpu.sync_copy(py_hbm, py_v)
    pltpu.sync_copy(pz_hbm, pz_v)
    pltpu.sync_copy(src_hbm.at[pl.ds(base, EW)], src_v)
    pltpu.sync_copy(dst_hbm.at[pl.ds(base, EW)], dst_v)

    def body(i, carry):
        off = i * 16
        s16 = src_v[pl.ds(off, 16)]
        t16 = dst_v[pl.ds(off, 16)]
        ax = plsc.load_gather(px_v, [s16]) - plsc.load_gather(px_v, [t16])
        ay = plsc.load_gather(py_v, [s16]) - plsc.load_gather(py_v, [t16])
        az = plsc.load_gather(pz_v, [s16]) - plsc.load_gather(pz_v, [t16])
        d2_v[pl.ds(off, 16)] = ax * ax + ay * ay + az * az + 1e-12
        return carry

    lax.fori_loop(0, EW // 16, body, 0)
    pltpu.sync_copy(d2_v, d2_hbm.at[pl.ds(base, EW)])


def _filter_body(d2_ref, wf1t_ref, bf1_ref, wf2_ref, bf2_ref, o_ref):
    d = jnp.sqrt(d2_ref[0])
    mu = lax.broadcasted_iota(jnp.int32, (F, 1), 0).astype(jnp.float32) * STEP
    diff = d - mu
    rbft = jnp.exp(diff * diff * (-GAMMA))
    h = _ssp(
        jnp.dot(wf1t_ref[...], rbft, preferred_element_type=jnp.float32)
        + bf1_ref[...]
    )
    w = lax.dot_general(
        h, wf2_ref[...], (((0,), (0,)), ((), ())),
        preferred_element_type=jnp.float32,
    )
    o_ref[...] = w + bf2_ref[...]


@functools.lru_cache(maxsize=None)
def _sc_gms_kernel():
    return functools.partial(
        pl.kernel,
        out_type=[
            jax.ShapeDtypeStruct((NC, N, F), jnp.float32),
            jax.ShapeDtypeStruct((NW, EW, F), jnp.float32),
        ],
        mesh=_mesh(),
        compiler_params=pltpu.CompilerParams(needs_layout_passes=False),
        scratch_types=[
            pltpu.VMEM((EW,), jnp.int32),
            pltpu.VMEM((EW,), jnp.int32),
            pltpu.VMEM((2, K), jnp.int32),
            pltpu.VMEM((K, F), jnp.float32),
            pltpu.VMEM((K, F), jnp.float32),
            pltpu.VMEM((K, F), jnp.float32),
            pltpu.VMEM((K, F), jnp.float32),
            pltpu.VMEM_SHARED((NPH + TRASH, F), jnp.float32),
            pltpu.SemaphoreType.DMA,
            pltpu.SemaphoreType.DMA,
            pltpu.SemaphoreType.DMA,
            pltpu.SemaphoreType.DMA,
            pltpu.SemaphoreType.DMA,
            pltpu.SemaphoreType.DMA,
        ],
    )(_sc_gms)


def _sc_gms(m_hbm, w_hbm, edges_hbm, vp_hbm, spill_hbm,
            src_v, dst_v, ixs_v, rows_0, rows_1, w_0, w_1, vacc,
            gsem_0, gsem_1, wsem_0, wsem_1, psem_0, psem_1):
    cid = lax.axis_index("c")
    sid = lax.axis_index("s")
    wid = sid * NC + cid
    pltpu.sync_copy(edges_hbm.at[0, wid], src_v)
    pltpu.sync_copy(edges_hbm.at[1, wid], dst_v)
    gbase = wid * EW

    rows_b = (rows_0, rows_1)
    w_b = (w_0, w_1)
    gsem_b = (gsem_0, gsem_1)
    wsem_b = (wsem_0, wsem_1)
    psem_b = (psem_0, psem_1)

    def _zero_acc():
        def zrow(r, carry):
            for cc in range(F // 16):
                rows_0[r, pl.ds(cc * 16, 16)] = jnp.zeros((16,), jnp.float32)
            return carry

        lax.fori_loop(0, K, zrow, 0)

        def zcopy(t, carry):
            pltpu.sync_copy(rows_0.at[pl.ds(0, 40)],
                            vacc.at[pl.ds(sid * PPT + t * 40, 40)])
            return carry

        lax.fori_loop(0, PPT // 40, zcopy, 0)

        @pl.when(sid == NS - 1)
        def _zero_trash():
            pltpu.sync_copy(rows_0.at[pl.ds(0, TRASH)],
                            vacc.at[pl.ds(NPH, TRASH)])

    def _ixcalc(j, b, base):
        for g in range(K // 16):
            sl = pl.ds(g * 16, 16)
            rel = dst_v[pl.ds(j * K + g * 16, 16)] - base
            inr = jnp.logical_and(rel >= 0, rel < NPH)
            low = jnp.bitwise_and(rel, TRASH - 1)
            ixs_v[b, sl] = jnp.where(inr, rel, NPH + low)

    def _dump(base, nph_real):
        full_tiles = nph_real // PPT
        rem = nph_real - full_tiles * PPT

        @pl.when(sid < full_tiles)
        def _dump_full():
            pltpu.sync_copy(vacc.at[pl.ds(sid * PPT, PPT)],
                            vp_hbm.at[cid, pl.ds(base + sid * PPT, PPT)])

        if rem:
            @pl.when(sid == full_tiles)
            def _dump_rem():
                pltpu.sync_copy(
                    vacc.at[pl.ds(full_tiles * PPT, rem)],
                    vp_hbm.at[cid, pl.ds(base + full_tiles * PPT, rem)])

    def _spill_desc(j, b):
        return pltpu.make_async_copy(
            rows_b[b], spill_hbm.at[wid, pl.ds(j * K, K)], psem_b[b])

    def _gather_desc(j, b):
        return pltpu.make_async_copy(
            m_hbm.at[src_v.at[pl.ds(j * K, K)]], rows_b[b], gsem_b[b])

    def _w_desc(j, b):
        return pltpu.make_async_copy(
            w_hbm.at[pl.ds(gbase + j * K, K)], w_b[b], wsem_b[b])

    def _start1(j, b):
        _gather_desc(j, b).start()
        _w_desc(j, b).start()

    def _process1(j, b):
        _ixcalc(j, b, 0)
        _gather_desc(j, b).wait()
        _w_desc(j, b).wait()

        def mul(r, c2):
            for cc in range(F // 16):
                sl = pl.ds(cc * 16, 16)
                rows_b[b][r, sl] = rows_b[b][r, sl] * w_b[b][r, sl]
            return c2

        lax.fori_loop(0, K, mul, 0)
        _spill_desc(j, b).start()
        pltpu.sync_copy(rows_b[b], vacc.at[ixs_v.at[b]], add=True)

    _zero_acc()
    plsc.subcore_barrier()

    _start1(0, 0)
    _start1(1, 1)

    def pair1(jj, carry):
        j0 = jj * 2
        _process1(j0, 0)
        _spill_desc(j0, 0).wait()
        _start1(j0 + 2, 0)
        _process1(j0 + 1, 1)

        @pl.when(jj < C // 2 - 1)
        def _next():
            _spill_desc(j0 + 1, 1).wait()
            _start1(j0 + 3, 1)

        return carry

    lax.fori_loop(0, C // 2, pair1, 0)
    _process1(C - 1, 0)
    _spill_desc(C - 2, 1).wait()
    _spill_desc(C - 1, 0).wait()
    plsc.subcore_barrier()
    _dump(0, NPH)
    plsc.subcore_barrier()

    def _rd_desc(j, b):
        return pltpu.make_async_copy(
            spill_hbm.at[wid, pl.ds(j * K, K)], rows_b[b], gsem_b[b])

    def _process2(j, b):
        _ixcalc(j, b, NPH)
        _rd_desc(j, b).wait()
        pltpu.sync_copy(rows_b[b], vacc.at[ixs_v.at[b]], add=True)

    _zero_acc()
    plsc.subcore_barrier()
    _rd_desc(0, 0).start()
    _rd_desc(1, 1).start()

    def pair2(jj, carry):
        j0 = jj * 2
        _process2(j0, 0)
        _rd_desc(j0 + 2, 0).start()
        _process2(j0 + 1, 1)

        @pl.when(jj < C // 2 - 1)
        def _next():
            _rd_desc(j0 + 3, 1).start()

        return carry

    lax.fori_loop(0, C // 2, pair2, 0)
    _process2(C - 1, 0)
    plsc.subcore_barrier()
    _dump(NPH, N - NPH)
    plsc.subcore_barrier()


def _out_body(x_ref, vp_ref, w2_ref, b2_ref, w3_ref, b3_ref, o_ref):
    v = vp_ref[0] + vp_ref[1]
    t = _ssp(
        jnp.dot(v, w2_ref[...], preferred_element_type=jnp.float32)
        + b2_ref[...]
    )
    o_ref[...] = (
        x_ref[...]
        + jnp.dot(t, w3_ref[...], preferred_element_type=jnp.float32)
        + b3_ref[...]
    )


def kernel(x, edge_index, z, position, W1, b1, Wf1, bf1, Wf2, bf2, W2, b2, W3, b3):
    del z
    src = edge_index[0]
    dst = edge_index[1]
    px = position[:, 0]
    py = position[:, 1]
    pz = position[:, 2]

    m = pl.pallas_call(
        _mm1_body,
        out_shape=jax.ShapeDtypeStruct((N, F), jnp.float32),
    )(x, W1, b1.reshape(1, F))

    d2 = _sc_d2_kernel()(px, py, pz, src, dst)

    wf1t = jnp.zeros((F, F), jnp.float32).at[:, :N_RBF].set(Wf1.T)
    w = pl.pallas_call(
        _filter_body,
        grid=(NB,),
        in_specs=[
            pl.BlockSpec((1, 1, BE), lambda i: (i, 0, 0)),
            pl.BlockSpec((F, F), lambda i: (0, 0)),
            pl.BlockSpec((F, 1), lambda i: (0, 0)),
            pl.BlockSpec((F, F), lambda i: (0, 0)),
            pl.BlockSpec((1, F), lambda i: (0, 0)),
        ],
        out_specs=pl.BlockSpec((BE, F), lambda i: (i, 0)),
        out_shape=jax.ShapeDtypeStruct((E, F), jnp.float32),
    )(d2.reshape(NB, 1, BE), wf1t, bf1.reshape(F, 1), Wf2, bf2.reshape(1, F))

    edges = edge_index.reshape(2, NW, EW)
    vp, _spill = _sc_gms_kernel()(m, w, edges)

    out = pl.pallas_call(
        _out_body,
        out_shape=jax.ShapeDtypeStruct((N, F), jnp.float32),
    )(x, vp, W2, b2.reshape(1, F), W3, b3.reshape(1, F))
    return out


# speedup vs baseline: 6.0204x; 1.0167x over previous
import functools

import jax
import jax.numpy as jnp
from jax import lax
from jax.experimental import pallas as pl
from jax.experimental.pallas import tpu as pltpu
from jax.experimental.pallas import tpu_sc as plsc

N = 10000
E = 320000
F = 128
N_RBF = 100
GAMMA = 10.0
STEP = 0.1
LN2 = 0.6931471805599453

NC = 2
NS = 16
NW = NC * NS
EW = E // NW
K = 80
C = EW // K

NPH = 5120
TRASH = 8
PPT = NPH // NS

BE = 2560
NB = E // BE
NMU = 104


@functools.lru_cache(maxsize=None)
def _mesh():
    return plsc.VectorSubcoreMesh(
        core_axis_name="c", subcore_axis_name="s",
        num_cores=NC, num_subcores=NS,
    )


def _ssp(t):
    return jnp.logaddexp(t, 0.0) - LN2


def _mm1_body(x_ref, w_ref, b_ref, o_ref):
    o_ref[...] = (
        jnp.dot(x_ref[...], w_ref[...], preferred_element_type=jnp.float32)
        + b_ref[...]
    )


@functools.lru_cache(maxsize=None)
def _sc_d2_kernel():
    return functools.partial(
        pl.kernel,
        out_type=jax.ShapeDtypeStruct((E,), jnp.float32),
        mesh=_mesh(),
        compiler_params=pltpu.CompilerParams(needs_layout_passes=False),
        scratch_types=[
            pltpu.VMEM((N,), jnp.float32),
            pltpu.VMEM((N,), jnp.float32),
            pltpu.VMEM((N,), jnp.float32),
            pltpu.VMEM((EW,), jnp.int32),
            pltpu.VMEM((EW,), jnp.int32),
            pltpu.VMEM((EW,), jnp.float32),
        ],
    )(_sc_d2)


def _sc_d2(px_hbm, py_hbm, pz_hbm, src_hbm, dst_hbm, d2_hbm,
           px_v, py_v, pz_v, src_v, dst_v, d2_v):
    cid = lax.axis_index("c")
    sid = lax.axis_index("s")
    wid = sid * NC + cid
    base = wid * EW
    pltpu.sync_copy(px_hbm, px_v)
    pltpu.sync_copy(py_hbm, py_v)
    pltpu.sync_copy(pz_hbm, pz_v)
    pltpu.sync_copy(src_hbm.at[pl.ds(base, EW)], src_v)
    pltpu.sync_copy(dst_hbm.at[pl.ds(base, EW)], dst_v)

    def body(i, carry):
        off = i * 16
        s16 = src_v[pl.ds(off, 16)]
        t16 = dst_v[pl.ds(off, 16)]
        ax = plsc.load_gather(px_v, [s16]) - plsc.load_gather(px_v, [t16])
        ay = plsc.load_gather(py_v, [s16]) - plsc.load_gather(py_v, [t16])
        az = plsc.load_gather(pz_v, [s16]) - plsc.load_gather(pz_v, [t16])
        d2_v[pl.ds(off, 16)] = ax * ax + ay * ay + az * az + 1e-12
        return carry

    lax.fori_loop(0, EW // 16, body, 0)
    pltpu.sync_copy(d2_v, d2_hbm.at[pl.ds(base, EW)])


def _filter_body(d2_ref, wf1t_ref, bf1_ref, wf2_ref, bf2_ref, o_ref):
    d = jnp.sqrt(d2_ref[0])
    mu = lax.broadcasted_iota(jnp.int32, (NMU, 1), 0).astype(jnp.float32) * STEP
    diff = d - mu
    rbft = jnp.exp(diff * diff * (-GAMMA))
    h = _ssp(
        jnp.dot(wf1t_ref[...], rbft, preferred_element_type=jnp.float32)
        + bf1_ref[...]
    )
    w = lax.dot_general(
        h, wf2_ref[...], (((0,), (0,)), ((), ())),
        preferred_element_type=jnp.float32,
    )
    o_ref[...] = w + bf2_ref[...]


@functools.lru_cache(maxsize=None)
def _sc_gms_kernel():
    return functools.partial(
        pl.kernel,
        out_type=[
            jax.ShapeDtypeStruct((NC, N, F), jnp.float32),
            jax.ShapeDtypeStruct((NW, EW, F), jnp.float32),
        ],
        mesh=_mesh(),
        compiler_params=pltpu.CompilerParams(needs_layout_passes=False),
        scratch_types=[
            pltpu.VMEM((EW,), jnp.int32),
            pltpu.VMEM((EW,), jnp.int32),
            pltpu.VMEM((2, K), jnp.int32),
            pltpu.VMEM((K, F), jnp.float32),
            pltpu.VMEM((K, F), jnp.float32),
            pltpu.VMEM((K, F), jnp.float32),
            pltpu.VMEM((K, F), jnp.float32),
            pltpu.VMEM_SHARED((NPH + TRASH, F), jnp.float32),
            pltpu.SemaphoreType.DMA,
            pltpu.SemaphoreType.DMA,
            pltpu.SemaphoreType.DMA,
            pltpu.SemaphoreType.DMA,
            pltpu.SemaphoreType.DMA,
            pltpu.SemaphoreType.DMA,
        ],
    )(_sc_gms)


def _sc_gms(m_hbm, w_hbm, edges_hbm, vp_hbm, spill_hbm,
            src_v, dst_v, ixs_v, rows_0, rows_1, w_0, w_1, vacc,
            gsem_0, gsem_1, wsem_0, wsem_1, psem_0, psem_1):
    cid = lax.axis_index("c")
    sid = lax.axis_index("s")
    wid = sid * NC + cid
    pltpu.sync_copy(edges_hbm.at[0, wid], src_v)
    pltpu.sync_copy(edges_hbm.at[1, wid], dst_v)
    gbase = wid * EW

    rows_b = (rows_0, rows_1)
    w_b = (w_0, w_1)
    gsem_b = (gsem_0, gsem_1)
    wsem_b = (wsem_0, wsem_1)
    psem_b = (psem_0, psem_1)

    def _zero_acc():
        def zrow(r, carry):
            for cc in range(F // 16):
                rows_0[r, pl.ds(cc * 16, 16)] = jnp.zeros((16,), jnp.float32)
            return carry

        lax.fori_loop(0, K, zrow, 0)

        def zcopy(t, carry):
            pltpu.sync_copy(rows_0.at[pl.ds(0, 40)],
                            vacc.at[pl.ds(sid * PPT + t * 40, 40)])
            return carry

        lax.fori_loop(0, PPT // 40, zcopy, 0)

        @pl.when(sid == NS - 1)
        def _zero_trash():
            pltpu.sync_copy(rows_0.at[pl.ds(0, TRASH)],
                            vacc.at[pl.ds(NPH, TRASH)])

    def _ixcalc(j, b, base):
        for g in range(K // 16):
            sl = pl.ds(g * 16, 16)
            rel = dst_v[pl.ds(j * K + g * 16, 16)] - base
            inr = jnp.logical_and(rel >= 0, rel < NPH)
            low = jnp.bitwise_and(rel, TRASH - 1)
            ixs_v[b, sl] = jnp.where(inr, rel, NPH + low)

    def _dump(base, nph_real):
        full_tiles = nph_real // PPT
        rem = nph_real - full_tiles * PPT

        @pl.when(sid < full_tiles)
        def _dump_full():
            pltpu.sync_copy(vacc.at[pl.ds(sid * PPT, PPT)],
                            vp_hbm.at[cid, pl.ds(base + sid * PPT, PPT)])

        if rem:
            @pl.when(sid == full_tiles)
            def _dump_rem():
                pltpu.sync_copy(
                    vacc.at[pl.ds(full_tiles * PPT, rem)],
                    vp_hbm.at[cid, pl.ds(base + full_tiles * PPT, rem)])

    def _spill_desc(j, b):
        return pltpu.make_async_copy(
            rows_b[b], spill_hbm.at[wid, pl.ds(j * K, K)], psem_b[b])

    def _gather_desc(j, b):
        return pltpu.make_async_copy(
            m_hbm.at[src_v.at[pl.ds(j * K, K)]], rows_b[b], gsem_b[b])

    def _w_desc(j, b):
        return pltpu.make_async_copy(
            w_hbm.at[pl.ds(gbase + j * K, K)], w_b[b], wsem_b[b])

    def _start1(j, b):
        _gather_desc(j, b).start()
        _w_desc(j, b).start()

    def _process1(j, b):
        _ixcalc(j, b, 0)
        _gather_desc(j, b).wait()
        _w_desc(j, b).wait()

        def mul(r, c2):
            for cc in range(F // 16):
                sl = pl.ds(cc * 16, 16)
                rows_b[b][r, sl] = rows_b[b][r, sl] * w_b[b][r, sl]
            return c2

        lax.fori_loop(0, K, mul, 0)
        _spill_desc(j, b).start()
        pltpu.sync_copy(rows_b[b], vacc.at[ixs_v.at[b]], add=True)

    _zero_acc()
    plsc.subcore_barrier()

    _start1(0, 0)
    _start1(1, 1)

    def pair1(jj, carry):
        j0 = jj * 2
        _process1(j0, 0)
        _spill_desc(j0, 0).wait()
        _start1(j0 + 2, 0)
        _process1(j0 + 1, 1)

        @pl.when(jj < C // 2 - 1)
        def _next():
            _spill_desc(j0 + 1, 1).wait()
            _start1(j0 + 3, 1)

        return carry

    lax.fori_loop(0, C // 2, pair1, 0)
    _process1(C - 1, 0)
    _spill_desc(C - 2, 1).wait()
    _spill_desc(C - 1, 0).wait()
    plsc.subcore_barrier()
    _dump(0, NPH)
    plsc.subcore_barrier()

    def _rd_desc(j, b):
        return pltpu.make_async_copy(
            spill_hbm.at[wid, pl.ds(j * K, K)], rows_b[b], gsem_b[b])

    def _process2(j, b):
        _ixcalc(j, b, NPH)
        _rd_desc(j, b).wait()
        pltpu.sync_copy(rows_b[b], vacc.at[ixs_v.at[b]], add=True)

    _zero_acc()
    plsc.subcore_barrier()
    _rd_desc(0, 0).start()
    _rd_desc(1, 1).start()

    def pair2(jj, carry):
        j0 = jj * 2
        _process2(j0, 0)
        _rd_desc(j0 + 2, 0).start()
        _process2(j0 + 1, 1)

        @pl.when(jj < C // 2 - 1)
        def _next():
            _rd_desc(j0 + 3, 1).start()

        return carry

    lax.fori_loop(0, C // 2, pair2, 0)
    _process2(C - 1, 0)
    plsc.subcore_barrier()
    _dump(NPH, N - NPH)
    plsc.subcore_barrier()


def _out_body(x_ref, vp_ref, w2_ref, b2_ref, w3_ref, b3_ref, o_ref):
    v = vp_ref[0] + vp_ref[1]
    t = _ssp(
        jnp.dot(v, w2_ref[...], preferred_element_type=jnp.float32)
        + b2_ref[...]
    )
    o_ref[...] = (
        x_ref[...]
        + jnp.dot(t, w3_ref[...], preferred_element_type=jnp.float32)
        + b3_ref[...]
    )


def kernel(x, edge_index, z, position, W1, b1, Wf1, bf1, Wf2, bf2, W2, b2, W3, b3):
    del z
    src = edge_index[0]
    dst = edge_index[1]
    px = position[:, 0]
    py = position[:, 1]
    pz = position[:, 2]

    m = pl.pallas_call(
        _mm1_body,
        out_shape=jax.ShapeDtypeStruct((N, F), jnp.float32),
    )(x, W1, b1.reshape(1, F))

    d2 = _sc_d2_kernel()(px, py, pz, src, dst)

    wf1t = jnp.zeros((F, NMU), jnp.float32).at[:, :N_RBF].set(Wf1.T)
    w = pl.pallas_call(
        _filter_body,
        grid=(NB,),
        in_specs=[
            pl.BlockSpec((1, 1, BE), lambda i: (i, 0, 0)),
            pl.BlockSpec((F, NMU), lambda i: (0, 0)),
            pl.BlockSpec((F, 1), lambda i: (0, 0)),
            pl.BlockSpec((F, F), lambda i: (0, 0)),
            pl.BlockSpec((1, F), lambda i: (0, 0)),
        ],
        out_specs=pl.BlockSpec((BE, F), lambda i: (i, 0)),
        out_shape=jax.ShapeDtypeStruct((E, F), jnp.float32),
    )(d2.reshape(NB, 1, BE), wf1t, bf1.reshape(F, 1), Wf2, bf2.reshape(1, F))

    edges = edge_index.reshape(2, NW, EW)
    vp, _spill = _sc_gms_kernel()(m, w, edges)

    out = pl.pallas_call(
        _out_body,
        out_shape=jax.ShapeDtypeStruct((N, F), jnp.float32),
    )(x, vp, W2, b2.reshape(1, F), W3, b3.reshape(1, F))
    return out
